# baseline (speedup 1.0000x reference)
import functools

import jax
import jax.numpy as jnp
from jax import lax
from jax.experimental import pallas as pl
from jax.experimental.pallas import tpu as pltpu

N = 50000
E = 800000
FEAT = 128
HID = 64

_BN = 2000
_BE = 6400


def _h0_body(x_ref, w_ref, b_ref, o_ref):
    o_ref[...] = jax.nn.relu(
        jnp.dot(x_ref[...], w_ref[...], preferred_element_type=jnp.float32)
        + b_ref[...]
    )


def _h0(x, W_in, b_in):
    return pl.pallas_call(
        _h0_body,
        grid=(N // _BN,),
        in_specs=[
            pl.BlockSpec((_BN, FEAT), lambda i: (i, 0)),
            pl.BlockSpec((FEAT, HID), lambda i: (0, 0)),
            pl.BlockSpec((1, HID), lambda i: (0, 0)),
        ],
        out_specs=pl.BlockSpec((_BN, HID), lambda i: (i, 0)),
        out_shape=jax.ShapeDtypeStruct((N, HID), jnp.float32),
    )(x, W_in, b_in.reshape(1, HID))


def _edgew_body(ea_ref, we1_ref, be1_ref, we2_ref, be2_ref, o_ref):
    ea = ea_ref[...]
    for l in range(3):
        h16 = jax.nn.relu(
            jnp.dot(ea, we1_ref[l], preferred_element_type=jnp.float32)
            + be1_ref[l]
        )
        w = jax.nn.sigmoid(
            jnp.dot(h16, we2_ref[l], preferred_element_type=jnp.float32)
            + be2_ref[l]
        )
        o_ref[l, :] = w[:, 0]


def _edge_w(edge_attr, We1, be1, We2, be2):
    return pl.pallas_call(
        _edgew_body,
        grid=(E // _BE,),
        in_specs=[
            pl.BlockSpec((_BE, 3), lambda i: (i, 0)),
            pl.BlockSpec((3, 3, 16), lambda i: (0, 0, 0)),
            pl.BlockSpec((3, 1, 16), lambda i: (0, 0, 0)),
            pl.BlockSpec((3, 16, 1), lambda i: (0, 0, 0)),
            pl.BlockSpec((3, 1, 1), lambda i: (0, 0, 0)),
        ],
        out_specs=pl.BlockSpec((3, _BE), lambda i: (0, i)),
        out_shape=jax.ShapeDtypeStruct((3, E), jnp.float32),
    )(edge_attr, We1, be1, We2, be2)


def _upd_body(h_ref, agg_ref, cnt_ref, wn_ref, bn_ref, g_ref, b_ref, o_ref):
    h = h_ref[...]
    cnt = jnp.clip(cnt_ref[...], 1e-12, None)
    agg = agg_ref[...] / cnt
    upd = jax.nn.relu(
        jnp.dot(h, wn_ref[0], preferred_element_type=jnp.float32)
        + jnp.dot(agg, wn_ref[1], preferred_element_type=jnp.float32)
        + bn_ref[...]
    )
    r = h + upd
    mu = jnp.mean(r, axis=-1, keepdims=True)
    var = jnp.mean((r - mu) ** 2, axis=-1, keepdims=True)
    o_ref[...] = (r - mu) * lax.rsqrt(var + 1e-5) * g_ref[...] + b_ref[...]


def _layer_update(h, agg, cnt, Wn, bn, g, b):
    return pl.pallas_call(
        _upd_body,
        grid=(N // _BN,),
        in_specs=[
            pl.BlockSpec((_BN, HID), lambda i: (i, 0)),
            pl.BlockSpec((_BN, HID), lambda i: (i, 0)),
            pl.BlockSpec((_BN, 1), lambda i: (i, 0)),
            pl.BlockSpec((2, HID, HID), lambda i: (0, 0, 0)),
            pl.BlockSpec((1, HID), lambda i: (0, 0)),
            pl.BlockSpec((1, HID), lambda i: (0, 0)),
            pl.BlockSpec((1, HID), lambda i: (0, 0)),
        ],
        out_specs=pl.BlockSpec((_BN, HID), lambda i: (i, 0)),
        out_shape=jax.ShapeDtypeStruct((N, HID), jnp.float32),
    )(h, agg, cnt, Wn, bn.reshape(1, HID), g.reshape(1, HID), b.reshape(1, HID))


def _head_body(h_ref, w1_ref, b1_ref, w2_ref, b2_ref, o_ref):
    z = jax.nn.relu(
        jnp.dot(h_ref[...], w1_ref[...], preferred_element_type=jnp.float32)
        + b1_ref[...]
    )
    o_ref[...] = (
        jnp.dot(z, w2_ref[...], preferred_element_type=jnp.float32) + b2_ref[...]
    )


def _head(h, Wh1, bh1, Wh2, bh2):
    return pl.pallas_call(
        _head_body,
        grid=(N // _BN,),
        in_specs=[
            pl.BlockSpec((_BN, HID), lambda i: (i, 0)),
            pl.BlockSpec((HID, HID // 2), lambda i: (0, 0)),
            pl.BlockSpec((1, HID // 2), lambda i: (0, 0)),
            pl.BlockSpec((HID // 2, 1), lambda i: (0, 0)),
            pl.BlockSpec((1, 1), lambda i: (0, 0)),
        ],
        out_specs=pl.BlockSpec((_BN, 1), lambda i: (i, 0)),
        out_shape=jax.ShapeDtypeStruct((N, 1), jnp.float32),
    )(h, Wh1, bh1.reshape(1, HID // 2), Wh2, bh2.reshape(1, 1))


def kernel(x, edge_index, edge_attr, params):
    src = edge_index[0]
    dst = edge_index[1]
    lp = params["layers"]

    We1 = jnp.stack([l["We1"] for l in lp])
    be1 = jnp.stack([l["be1"].reshape(1, 16) for l in lp])
    We2 = jnp.stack([l["We2"] for l in lp])
    be2 = jnp.stack([l["be2"].reshape(1, 1) for l in lp])

    w_all = _edge_w(edge_attr, We1, be1, We2, be2)
    h = _h0(x, params["W_in"], params["b_in"])

    for l in range(3):
        w = w_all[l][:, None]
        agg = jnp.zeros((N, HID), jnp.float32).at[dst].add(h[src] * w)
        cnt = jnp.zeros((N, 1), jnp.float32).at[dst].add(w)
        Wn = jnp.stack([lp[l]["Wn"][:HID], lp[l]["Wn"][HID:]])
        h = _layer_update(h, agg, cnt, Wn, lp[l]["bn"], lp[l]["ln_g"], lp[l]["ln_b"])

    return _head(h, params["Wh1"], params["bh1"], params["Wh2"], params["bh2"])
